# baseline (speedup 1.0000x reference)
import functools

import jax
import jax.numpy as jnp
from jax import lax
from jax.experimental import pallas as pl
from jax.experimental.pallas import tpu as pltpu
from jax.experimental.pallas import tpu_sc as plsc


@functools.lru_cache(maxsize=None)
def _make_kernel(B, L, E):
    info = plsc.get_sparse_core_info()
    NC, NS, LANES = info.num_cores, info.num_subcores, info.num_lanes
    NW = NC * NS
    total = B * L
    assert total % (NW * L) == 0
    rows_per_w = total // NW
    seqs_per_w = rows_per_w // L
    CH0 = 128
    CH1 = L - CH0

    mesh = plsc.VectorSubcoreMesh(core_axis_name="c", subcore_axis_name="s")

    @functools.partial(
        pl.kernel,
        mesh=mesh,
        out_type=jax.ShapeDtypeStruct((total, E), jnp.float32),
        scratch_types=[
            pltpu.VMEM((L, E), jnp.float32),
            pltpu.VMEM((CH0,), jnp.int32),
            pltpu.VMEM((CH0, E), jnp.float32),
            pltpu.VMEM((CH1,), jnp.int32),
            pltpu.VMEM((CH1, E), jnp.float32),
            pltpu.SemaphoreType.DMA,
        ],
        compiler_params=pltpu.CompilerParams(use_tc_tiling_on_sc=False),
    )
    def k(x_hbm, tok_hbm, pos_hbm, out_hbm, pos_v, idx_a, rows_a, idx_b, rows_b, sem):
        wid = lax.axis_index("s") * NC + lax.axis_index("c")
        pltpu.sync_copy(pos_hbm, pos_v)
        base = wid * rows_per_w

        def seq_body(s, carry):
            row0 = base + s * L
            for off, n, idx_v, rows_v in (
                (0, CH0, idx_a, rows_a),
                (CH0, CH1, idx_b, rows_b),
            ):
                r0 = row0 + off
                pltpu.sync_copy(x_hbm.at[pl.ds(r0, n)], idx_v)
                pltpu.async_copy(tok_hbm.at[idx_v], rows_v, sem).wait()

                def add_body(r, c2, off=off, rows_v=rows_v):
                    for c in range(E // LANES):
                        sl = pl.ds(c * LANES, LANES)
                        plsc.addupdate(rows_v.at[r, sl], pos_v[off + r, sl])
                    return c2

                lax.fori_loop(0, n, add_body, 0)
                pltpu.sync_copy(rows_v, out_hbm.at[pl.ds(r0, n)])
            return carry

        lax.fori_loop(0, seqs_per_w, seq_body, 0)

    return k


def kernel(x, token_table, pos_table):
    B, L = x.shape
    V, E = token_table.shape
    k = _make_kernel(B, L, E)
    out = k(x.reshape(B * L), token_table, pos_table)
    return out.reshape(B, L, E)


# speedup vs baseline: 1.3182x; 1.3182x over previous
import functools

import jax
import jax.numpy as jnp
from jax import lax
from jax.experimental import pallas as pl
from jax.experimental.pallas import tpu as pltpu
from jax.experimental.pallas import tpu_sc as plsc

NBUF = 4


@functools.lru_cache(maxsize=None)
def _make_kernel(B, L, E):
    info = plsc.get_sparse_core_info()
    NC, NS, LANES = info.num_cores, info.num_subcores, info.num_lanes
    NW = NC * NS
    total = B * L
    assert total % (NW * L) == 0
    rows_per_w = total // NW
    seqs_per_w = rows_per_w // L
    assert seqs_per_w % NBUF == 0
    CH0 = 128
    CH1 = L - CH0
    UNROLL = 4
    assert L % UNROLL == 0

    mesh = plsc.VectorSubcoreMesh(core_axis_name="c", subcore_axis_name="s")

    @functools.partial(
        pl.kernel,
        mesh=mesh,
        out_type=jax.ShapeDtypeStruct((total, E), jnp.float32),
        scratch_types=[
            pltpu.VMEM((L, E), jnp.float32),
            pltpu.VMEM((rows_per_w,), jnp.int32),
        ]
        + [pltpu.VMEM((L, E), jnp.float32) for _ in range(NBUF)]
        + [pltpu.SemaphoreType.DMA for _ in range(2 * NBUF)],
        compiler_params=pltpu.CompilerParams(use_tc_tiling_on_sc=False),
    )
    def k(x_hbm, tok_hbm, pos_hbm, out_hbm, pos_v, idx_all, *bufs):
        rows = bufs[:NBUF]
        gsem = bufs[NBUF : 2 * NBUF]
        osem = bufs[2 * NBUF :]
        wid = lax.axis_index("s") * NC + lax.axis_index("c")
        base = wid * rows_per_w
        pltpu.sync_copy(pos_hbm, pos_v)
        pltpu.sync_copy(x_hbm.at[pl.ds(base, rows_per_w)], idx_all)

        def fire_gather(s, b):
            o = pl.multiple_of(s * L, 8)
            pltpu.async_copy(
                tok_hbm.at[idx_all.at[pl.ds(o, CH0)]],
                rows[b].at[pl.ds(0, CH0)],
                gsem[b],
            )
            o2 = pl.multiple_of(s * L + CH0, 8)
            pltpu.async_copy(
                tok_hbm.at[idx_all.at[pl.ds(o2, CH1)]],
                rows[b].at[pl.ds(CH0, CH1)],
                gsem[b],
            )

        def wait_gather(b):
            pltpu.make_async_copy(
                tok_hbm.at[idx_all.at[pl.ds(0, CH0)]],
                rows[b].at[pl.ds(0, CH0)],
                gsem[b],
            ).wait()
            pltpu.make_async_copy(
                tok_hbm.at[idx_all.at[pl.ds(0, CH1)]],
                rows[b].at[pl.ds(CH0, CH1)],
                gsem[b],
            ).wait()

        def fire_out(s, b):
            o = pl.multiple_of(base + s * L, 8)
            pltpu.async_copy(rows[b], out_hbm.at[pl.ds(o, L)], osem[b])

        def wait_out(b):
            pltpu.make_async_copy(rows[b], out_hbm.at[pl.ds(base, L)], osem[b]).wait()

        def add_pos(b):
            def body(i, carry):
                for u in range(UNROLL):
                    r = i * UNROLL + u
                    for c in range(E // LANES):
                        sl = pl.ds(c * LANES, LANES)
                        plsc.addupdate(rows[b].at[r, sl], pos_v[r, sl])
                return carry

            lax.fori_loop(0, L // UNROLL, body, 0)

        for b in range(NBUF - 1):
            fire_gather(b, b)

        def outer(k0, carry):
            g0 = k0 * NBUF
            for b in range(NBUF):
                g = g0 + b
                wait_gather(b)
                add_pos(b)
                fire_out(g, b)
                pb = (b - 1) % NBUF

                @pl.when(g >= 1)
                def _():
                    wait_out(pb)

                @pl.when(g + NBUF - 1 < seqs_per_w)
                def _():
                    fire_gather(g + NBUF - 1, pb)

            return carry

        lax.fori_loop(0, seqs_per_w // NBUF, outer, 0)
        wait_out(NBUF - 1)

    return k


def kernel(x, token_table, pos_table):
    B, L = x.shape
    V, E = token_table.shape
    k = _make_kernel(B, L, E)
    out = k(x.reshape(B * L), token_table, pos_table)
    return out.reshape(B, L, E)


# speedup vs baseline: 1.3212x; 1.0023x over previous
import functools

import jax
import jax.numpy as jnp
from jax import lax
from jax.experimental import pallas as pl
from jax.experimental.pallas import tpu as pltpu
from jax.experimental.pallas import tpu_sc as plsc

NBUF = 4


@functools.lru_cache(maxsize=None)
def _make_kernel(B, L, E):
    info = plsc.get_sparse_core_info()
    NC, NS, LANES = info.num_cores, info.num_subcores, info.num_lanes
    NW = NC * NS
    total = B * L
    assert total % (NW * L) == 0
    rows_per_w = total // NW
    seqs_per_w = rows_per_w // L
    assert seqs_per_w % NBUF == 0
    UNROLL = 4
    assert L % UNROLL == 0

    mesh = plsc.VectorSubcoreMesh(core_axis_name="c", subcore_axis_name="s")

    @functools.partial(
        pl.kernel,
        mesh=mesh,
        out_type=jax.ShapeDtypeStruct((total, E), jnp.float32),
        scratch_types=[
            pltpu.VMEM((L, E), jnp.float32),
            pltpu.VMEM((rows_per_w,), jnp.int32),
        ]
        + [pltpu.VMEM((L, E), jnp.float32) for _ in range(NBUF)]
        + [pltpu.SemaphoreType.DMA for _ in range(2 * NBUF)],
        compiler_params=pltpu.CompilerParams(use_tc_tiling_on_sc=False),
    )
    def k(x_hbm, tok_hbm, pos_hbm, out_hbm, pos_v, idx_all, *bufs):
        rows = bufs[:NBUF]
        gsem = bufs[NBUF : 2 * NBUF]
        osem = bufs[2 * NBUF :]
        wid = lax.axis_index("s") * NC + lax.axis_index("c")
        base = wid * rows_per_w
        pltpu.sync_copy(pos_hbm, pos_v)
        pltpu.sync_copy(x_hbm.at[pl.ds(base, rows_per_w)], idx_all)

        def fire_gather(s, b):
            o = pl.multiple_of(s * L, 8)
            pltpu.async_copy(
                tok_hbm.at[idx_all.at[pl.ds(o, L)]], rows[b], gsem[b]
            )

        def wait_gather(b):
            pltpu.make_async_copy(
                tok_hbm.at[idx_all.at[pl.ds(0, L)]], rows[b], gsem[b]
            ).wait()

        def fire_out(s, b):
            o = pl.multiple_of(base + s * L, 8)
            pltpu.async_copy(rows[b], out_hbm.at[pl.ds(o, L)], osem[b])

        def wait_out(b):
            pltpu.make_async_copy(rows[b], out_hbm.at[pl.ds(base, L)], osem[b]).wait()

        def add_pos(b):
            def body(i, carry):
                for u in range(UNROLL):
                    r = i * UNROLL + u
                    for c in range(E // LANES):
                        sl = pl.ds(c * LANES, LANES)
                        plsc.addupdate(rows[b].at[r, sl], pos_v[r, sl])
                return carry

            lax.fori_loop(0, L // UNROLL, body, 0)

        for b in range(NBUF - 1):
            fire_gather(b, b)

        def outer(k0, carry):
            g0 = k0 * NBUF
            for b in range(NBUF):
                g = g0 + b
                wait_gather(b)
                add_pos(b)
                fire_out(g, b)
                pb = (b - 1) % NBUF

                @pl.when(g >= 1)
                def _():
                    wait_out(pb)

                @pl.when(g + NBUF - 1 < seqs_per_w)
                def _():
                    fire_gather(g + NBUF - 1, pb)

            return carry

        lax.fori_loop(0, seqs_per_w // NBUF, outer, 0)
        wait_out(NBUF - 1)

    return k


def kernel(x, token_table, pos_table):
    B, L = x.shape
    V, E = token_table.shape
    k = _make_kernel(B, L, E)
    out = k(x.reshape(B * L), token_table, pos_table)
    return out.reshape(B, L, E)
